# baseline (speedup 1.0000x reference)
import functools
import math

import jax
import jax.numpy as jnp
from jax import lax
from jax.experimental import pallas as pl
from jax.experimental.pallas import tpu as pltpu, tpu_sc as plsc

N = 10000
D_IN = 128
H = 128
OUT = 64
E = 320000
NC = 2
NS = 16
K = 128
CH = 160
EPW = CH * K
EP = NS * EPW
ACC_ROWS = 10240
RPW = ACC_ROWS // NS
GARBAGE = N
CNTW = 8
INV_STD = float(1.0 / math.sqrt(1.0 + 1e-5))


def _sc_agg_body(with_cnt, D, *refs):
    if with_cnt:
        (ztab, srcf, dstf, zacc_in, zcnt_in, ones_in, agg_o, cnt_o,
         acc, cnt_sh, sidx_a, sidx_b, didx_a, didx_b, rows_a, rows_b,
         rows32, ones, sem_a, sem_b, isem_a, isem_b) = refs
    else:
        (ztab, srcf, dstf, zacc_in, agg_o,
         acc, sidx_a, sidx_b, didx_a, didx_b, rows_a, rows_b, rows32,
         sem_a, sem_b, isem_a, isem_b) = refs
        cnt_o = cnt_sh = ones = None

    c = lax.axis_index("c")
    s = lax.axis_index("s")
    row0 = s * RPW
    base = c * EP + s * EPW

    pltpu.sync_copy(zacc_in, acc.at[pl.ds(row0, RPW)])
    if with_cnt:
        pltpu.sync_copy(zcnt_in, cnt_sh.at[pl.ds(row0, RPW)])
        pltpu.sync_copy(ones_in, ones)
    plsc.subcore_barrier()

    def idx_start(j, sidx, didx, isem):
        off = base + j * K
        pltpu.async_copy(srcf.at[pl.ds(off, K)], sidx, isem)
        pltpu.async_copy(dstf.at[pl.ds(off, K)], didx, isem)

    def idx_wait(sidx, didx, isem):
        pltpu.make_async_copy(srcf.at[pl.ds(0, K)], sidx, isem).wait()
        pltpu.make_async_copy(dstf.at[pl.ds(0, K)], didx, isem).wait()

    def gather_start(sidx, rows16, sem):
        pltpu.async_copy(ztab.at[sidx], rows16, sem)

    def gather_wait(sidx, rows16, sem):
        pltpu.make_async_copy(ztab.at[sidx], rows16, sem).wait()

    def unpack_rows(rows16):
        U = 8

        def row_block(rb, _):
            for u in range(U):
                r = rb * U + u
                for cg in range(D // 32):
                    w = rows16[r, pl.ds(cg * 16, 16)]
                    lo = lax.bitcast_convert_type(w << 16, jnp.float32)
                    hi = lax.bitcast_convert_type(w & jnp.int32(-65536),
                                                  jnp.float32)
                    rows32[r, pl.ds(cg * 32, 16)] = lo
                    rows32[r, pl.ds(cg * 32 + 16, 16)] = hi
            return 0
        lax.fori_loop(0, K // U, row_block, 0)

    def scatter(didx):
        pltpu.sync_copy(rows32, acc.at[didx], add=True)
        if with_cnt:
            pltpu.sync_copy(ones, cnt_sh.at[didx], add=True)

    idx_start(0, sidx_a, didx_a, isem_a)
    idx_start(1, sidx_b, didx_b, isem_b)
    idx_wait(sidx_a, didx_a, isem_a)
    gather_start(sidx_a, rows_a, sem_a)

    def step(t, _):
        j0 = 2 * t
        idx_wait(sidx_b, didx_b, isem_b)
        gather_start(sidx_b, rows_b, sem_b)
        gather_wait(sidx_a, rows_a, sem_a)
        unpack_rows(rows_a)
        scatter(didx_a)

        @pl.when(j0 + 2 < CH)
        def _():
            idx_start(j0 + 2, sidx_a, didx_a, isem_a)
        gather_wait(sidx_b, rows_b, sem_b)
        unpack_rows(rows_b)
        scatter(didx_b)

        @pl.when(j0 + 3 < CH)
        def _():
            idx_start(j0 + 3, sidx_b, didx_b, isem_b)

        @pl.when(j0 + 2 < CH)
        def _():
            idx_wait(sidx_a, didx_a, isem_a)
            gather_start(sidx_a, rows_a, sem_a)
        return 0

    lax.fori_loop(0, CH // 2, step, 0)
    plsc.subcore_barrier()

    obase = c * N + row0

    @pl.when(s < NS - 1)
    def _():
        pltpu.sync_copy(acc.at[pl.ds(row0, RPW)], agg_o.at[pl.ds(obase, RPW)])
        if with_cnt:
            pltpu.sync_copy(cnt_sh.at[pl.ds(row0, RPW)],
                            cnt_o.at[pl.ds(obase, RPW)])

    last = N - (NS - 1) * RPW

    @pl.when(s == NS - 1)
    def _():
        pltpu.sync_copy(acc.at[pl.ds(row0, last)], agg_o.at[pl.ds(obase, last)])
        if with_cnt:
            pltpu.sync_copy(cnt_sh.at[pl.ds(row0, last)],
                            cnt_o.at[pl.ds(obase, last)])


def _make_sc_agg(with_cnt, D):
    mesh = plsc.VectorSubcoreMesh(core_axis_name="c", subcore_axis_name="s",
                                  num_cores=NC, num_subcores=NS)
    out_type = [jax.ShapeDtypeStruct((NC * N, D), jnp.float32)]
    scratch = [
        pltpu.VMEM_SHARED((ACC_ROWS, D), jnp.float32),
    ]
    if with_cnt:
        out_type.append(jax.ShapeDtypeStruct((NC * N, CNTW), jnp.float32))
        scratch.append(pltpu.VMEM_SHARED((ACC_ROWS, CNTW), jnp.float32))
    scratch += [
        pltpu.VMEM((K,), jnp.int32),
        pltpu.VMEM((K,), jnp.int32),
        pltpu.VMEM((K,), jnp.int32),
        pltpu.VMEM((K,), jnp.int32),
        pltpu.VMEM((K, D // 2), jnp.int32),
        pltpu.VMEM((K, D // 2), jnp.int32),
        pltpu.VMEM((K, D), jnp.float32),
    ]
    if with_cnt:
        scratch.append(pltpu.VMEM((K, CNTW), jnp.float32))
    scratch += [pltpu.SemaphoreType.DMA] * 4

    return pl.kernel(
        functools.partial(_sc_agg_body, with_cnt, D),
        out_type=tuple(out_type),
        mesh=mesh,
        scratch_types=tuple(scratch),
        compiler_params=pltpu.CompilerParams(use_tc_tiling_on_sc=False),
        name=f"sc_agg_{D}_{int(with_cnt)}",
    )


BR = 2000
GRID = N // BR


def _pre_order(D):
    order = [0] * D
    for c in range(D // 32):
        for m in range(16):
            order[32 * c + 2 * m] = 32 * c + m
            order[32 * c + 2 * m + 1] = 32 * c + 16 + m
    return jnp.array(order, jnp.int32)


def _k1_body(xu, xi, wpu, bpu, wpi, bpi, wl0ui, wl0iu, wr0ui, wr0iu,
             z0u, z0i, r0u, r0i):
    hu = jax.nn.relu(jnp.dot(xu[...], wpu[...],
                             preferred_element_type=jnp.float32) + bpu[...])
    hi = jax.nn.relu(jnp.dot(xi[...], wpi[...],
                             preferred_element_type=jnp.float32) + bpi[...])
    z0u[...] = jnp.dot(hu, wl0ui[...],
                       preferred_element_type=jnp.float32).astype(jnp.bfloat16)
    z0i[...] = jnp.dot(hi, wl0iu[...],
                       preferred_element_type=jnp.float32).astype(jnp.bfloat16)
    r0u[...] = jnp.dot(hu, wr0iu[...], preferred_element_type=jnp.float32)
    r0i[...] = jnp.dot(hi, wr0ui[...], preferred_element_type=jnp.float32)


def _k2_body(agg_i, agg_u, cnt_i, cnt_u, r0u, r0i,
             bl0ui, bl0iu, g0u, be0u, g0i, be0i,
             wl1ui, wl1iu, wr1ui, wr1iu,
             z1u, z1i, r1u, r1i):
    mean_u = agg_u[...] / jnp.maximum(cnt_u[:, 0:1], 1.0)
    mean_i = agg_i[...] / jnp.maximum(cnt_i[:, 0:1], 1.0)
    tu = mean_u + bl0iu[...] + r0u[...]
    ti = mean_i + bl0ui[...] + r0i[...]
    hu = jax.nn.relu(tu * (g0u[...] * INV_STD) + be0u[...])
    hi = jax.nn.relu(ti * (g0i[...] * INV_STD) + be0i[...])
    z1u[...] = jnp.dot(hu, wl1ui[...],
                       preferred_element_type=jnp.float32).astype(jnp.bfloat16)
    z1i[...] = jnp.dot(hi, wl1iu[...],
                       preferred_element_type=jnp.float32).astype(jnp.bfloat16)
    r1u[...] = jnp.dot(hu, wr1iu[...], preferred_element_type=jnp.float32)
    r1i[...] = jnp.dot(hi, wr1ui[...], preferred_element_type=jnp.float32)


def _k3_body(agg_i, agg_u, cnt_i, cnt_u, r1u, r1i,
             bl1ui, bl1iu, g1u, be1u, g1i, be1i,
             hu_o, hi_o):
    mean_u = agg_u[...] / jnp.maximum(cnt_u[:, 0:1], 1.0)
    mean_i = agg_i[...] / jnp.maximum(cnt_i[:, 0:1], 1.0)
    tu = mean_u + bl1iu[...] + r1u[...]
    ti = mean_i + bl1ui[...] + r1i[...]
    hu_o[...] = jax.nn.relu(tu * (g1u[...] * INV_STD) + be1u[...])
    hi_o[...] = jax.nn.relu(ti * (g1i[...] * INV_STD) + be1i[...])


def _row_spec(w):
    return pl.BlockSpec((BR, w), lambda b: (b, 0))


def _row_spec_hi(w):
    return pl.BlockSpec((BR, w), lambda b: (b + GRID, 0))


def _full_spec(r, c):
    return pl.BlockSpec((r, c), lambda b: (0, 0))


def kernel(x_user, x_item, Wp_u, bp_u, Wp_i, bp_i,
           Wl0_ui, bl0_ui, Wr0_ui, Wl0_iu, bl0_iu, Wr0_iu,
           g0_u, be0_u, g0_i, be0_i,
           Wl1_ui, bl1_ui, Wr1_ui, Wl1_iu, bl1_iu, Wr1_iu,
           g1_u, be1_u, g1_i, be1_i,
           edge_index_ui, edge_index_iu):
    f32 = jnp.float32

    pad = EP - E
    src_ui = edge_index_ui[0].astype(jnp.int32)
    dst_ui = edge_index_ui[1].astype(jnp.int32)
    src_iu = edge_index_iu[0].astype(jnp.int32)
    dst_iu = edge_index_iu[1].astype(jnp.int32)
    zpad = jnp.zeros((pad,), jnp.int32)
    gpad = jnp.full((pad,), GARBAGE, jnp.int32)
    src_flat = jnp.concatenate([src_ui, zpad, src_iu + N, zpad + N])
    dst_flat = jnp.concatenate([dst_ui, gpad, dst_iu, gpad])

    row1 = lambda v: v.reshape(1, -1)

    k1 = pl.pallas_call(
        _k1_body,
        grid=(GRID,),
        in_specs=[
            _row_spec(D_IN), _row_spec(D_IN),
            _full_spec(D_IN, H), _full_spec(1, H),
            _full_spec(D_IN, H), _full_spec(1, H),
            _full_spec(H, H), _full_spec(H, H),
            _full_spec(H, H), _full_spec(H, H),
        ],
        out_specs=[_row_spec(H)] * 4,
        out_shape=[jax.ShapeDtypeStruct((N, H), jnp.bfloat16)] * 2
        + [jax.ShapeDtypeStruct((N, H), f32)] * 2,
    )
    p128 = _pre_order(H)
    p64 = _pre_order(OUT)
    z0u, z0i, r0u, r0i = k1(
        x_user, x_item, Wp_u.T, row1(bp_u), Wp_i.T, row1(bp_i),
        Wl0_ui.T[:, p128], Wl0_iu.T[:, p128], Wr0_ui.T, Wr0_iu.T)

    ztab0 = jax.lax.bitcast_convert_type(
        jnp.concatenate([z0u, z0i], axis=0).reshape(2 * N, H // 2, 2),
        jnp.int32)
    zacc_h = jnp.zeros((RPW, H), f32)
    zacc_o = jnp.zeros((RPW, OUT), f32)
    zcnt = jnp.zeros((RPW, CNTW), f32)
    ones_c = jnp.ones((K, CNTW), f32)
    agg0, cnt = _make_sc_agg(True, H)(ztab0, src_flat, dst_flat,
                                      zacc_h, zcnt, ones_c)

    k2 = pl.pallas_call(
        _k2_body,
        grid=(GRID,),
        in_specs=[
            _row_spec(H), _row_spec_hi(H),
            _row_spec(CNTW), _row_spec_hi(CNTW),
            _row_spec(H), _row_spec(H),
            _full_spec(1, H), _full_spec(1, H),
            _full_spec(1, H), _full_spec(1, H),
            _full_spec(1, H), _full_spec(1, H),
            _full_spec(H, OUT), _full_spec(H, OUT),
            _full_spec(H, OUT), _full_spec(H, OUT),
        ],
        out_specs=[_row_spec(OUT)] * 4,
        out_shape=[jax.ShapeDtypeStruct((N, OUT), jnp.bfloat16)] * 2
        + [jax.ShapeDtypeStruct((N, OUT), f32)] * 2,
    )
    z1u, z1i, r1u, r1i = k2(
        agg0, agg0, cnt, cnt, r0u, r0i,
        row1(bl0_ui), row1(bl0_iu), row1(g0_u), row1(be0_u),
        row1(g0_i), row1(be0_i),
        Wl1_ui.T[:, p64], Wl1_iu.T[:, p64], Wr1_ui.T, Wr1_iu.T)

    ztab1 = jax.lax.bitcast_convert_type(
        jnp.concatenate([z1u, z1i], axis=0).reshape(2 * N, OUT // 2, 2),
        jnp.int32)
    (agg1,) = _make_sc_agg(False, OUT)(ztab1, src_flat, dst_flat, zacc_o)

    k3 = pl.pallas_call(
        _k3_body,
        grid=(GRID,),
        in_specs=[
            _row_spec(OUT), _row_spec_hi(OUT),
            _row_spec(CNTW), _row_spec_hi(CNTW),
            _row_spec(OUT), _row_spec(OUT),
            _full_spec(1, OUT), _full_spec(1, OUT),
            _full_spec(1, OUT), _full_spec(1, OUT),
            _full_spec(1, OUT), _full_spec(1, OUT),
        ],
        out_specs=[_row_spec(OUT)] * 2,
        out_shape=[jax.ShapeDtypeStruct((N, OUT), f32)] * 2,
    )
    h_u, h_i = k3(
        agg1, agg1, cnt, cnt, r1u, r1i,
        row1(bl1_ui), row1(bl1_iu), row1(g1_u), row1(be1_u),
        row1(g1_i), row1(be1_i))
    return (h_u, h_i)


# speedup vs baseline: 1.2155x; 1.2155x over previous
import functools
import math

import jax
import jax.numpy as jnp
from jax import lax
from jax.experimental import pallas as pl
from jax.experimental.pallas import tpu as pltpu, tpu_sc as plsc

N = 10000
D_IN = 128
H = 128
OUT = 64
E = 320000
NC = 2
NS = 16
K = 128
CH = 160
EPW = CH * K
EP = NS * EPW
ACC_ROWS = 10240
RPW = ACC_ROWS // NS
GARBAGE = N
CNTW = 8
INV_STD = float(1.0 / math.sqrt(1.0 + 1e-5))


def _sc_agg_body(with_cnt, D, *refs):
    if with_cnt:
        (ztab, srcf, dstf, zacc_in, zcnt_in, ones_in, agg_o, cnt_o,
         acc, cnt_sh, sidx_a, sidx_b, didx_a, didx_b, rows_a, rows_b,
         rows32, ones, sem_a, sem_b, isem_a, isem_b, dsem_a, dsem_b) = refs
    else:
        (ztab, srcf, dstf, zacc_in, agg_o,
         acc, sidx_a, sidx_b, didx_a, didx_b, rows_a, rows_b, rows32,
         sem_a, sem_b, isem_a, isem_b, dsem_a, dsem_b) = refs
        cnt_o = cnt_sh = ones = None

    c = lax.axis_index("c")
    s = lax.axis_index("s")
    row0 = s * RPW
    base = c * EP + s * EPW

    pltpu.sync_copy(zacc_in, acc.at[pl.ds(row0, RPW)])
    if with_cnt:
        pltpu.sync_copy(zcnt_in, cnt_sh.at[pl.ds(row0, RPW)])
        pltpu.sync_copy(ones_in, ones)
    plsc.subcore_barrier()

    def sidx_start(j, sidx, isem):
        pltpu.async_copy(srcf.at[pl.ds(base + j * K, K)], sidx, isem)

    def sidx_wait(sidx, isem):
        pltpu.make_async_copy(srcf.at[pl.ds(0, K)], sidx, isem).wait()

    def didx_start(j, didx, dsem):
        pltpu.async_copy(dstf.at[pl.ds(base + j * K, K)], didx, dsem)

    def didx_wait(didx, dsem):
        pltpu.make_async_copy(dstf.at[pl.ds(0, K)], didx, dsem).wait()

    def gather_start(sidx, rows16, sem):
        pltpu.async_copy(ztab.at[sidx], rows16, sem)

    def gather_wait(sidx, rows16, sem):
        pltpu.make_async_copy(ztab.at[sidx], rows16, sem).wait()

    def unpack_rows(rows16):
        U = 8

        def row_block(rb, _):
            for u in range(U):
                r = rb * U + u
                for cg in range(D // 32):
                    w = rows16[r, pl.ds(cg * 16, 16)]
                    lo = lax.bitcast_convert_type(w << 16, jnp.float32)
                    hi = lax.bitcast_convert_type(w & jnp.int32(-65536),
                                                  jnp.float32)
                    rows32[r, pl.ds(cg * 32, 16)] = lo
                    rows32[r, pl.ds(cg * 32 + 16, 16)] = hi
            return 0
        lax.fori_loop(0, K // U, row_block, 0)

    def scatter(didx):
        pltpu.sync_copy(rows32, acc.at[didx], add=True)
        if with_cnt:
            pltpu.sync_copy(ones, cnt_sh.at[didx], add=True)

    sidx_start(0, sidx_a, isem_a)
    didx_start(0, didx_a, dsem_a)
    sidx_start(1, sidx_b, isem_b)
    didx_start(1, didx_b, dsem_b)
    sidx_wait(sidx_a, isem_a)
    gather_start(sidx_a, rows_a, sem_a)

    def step(t, _):
        j0 = 2 * t
        sidx_wait(sidx_b, isem_b)
        gather_start(sidx_b, rows_b, sem_b)
        gather_wait(sidx_a, rows_a, sem_a)

        @pl.when(j0 + 2 < CH)
        def _():
            sidx_start(j0 + 2, sidx_a, isem_a)
        unpack_rows(rows_a)
        didx_wait(didx_a, dsem_a)
        scatter(didx_a)

        @pl.when(j0 + 2 < CH)
        def _():
            didx_start(j0 + 2, didx_a, dsem_a)
            sidx_wait(sidx_a, isem_a)
            gather_start(sidx_a, rows_a, sem_a)
        gather_wait(sidx_b, rows_b, sem_b)

        @pl.when(j0 + 3 < CH)
        def _():
            sidx_start(j0 + 3, sidx_b, isem_b)
        unpack_rows(rows_b)
        didx_wait(didx_b, dsem_b)
        scatter(didx_b)

        @pl.when(j0 + 3 < CH)
        def _():
            didx_start(j0 + 3, didx_b, dsem_b)
        return 0

    lax.fori_loop(0, CH // 2, step, 0)
    plsc.subcore_barrier()

    obase = c * N + row0

    @pl.when(s < NS - 1)
    def _():
        pltpu.sync_copy(acc.at[pl.ds(row0, RPW)], agg_o.at[pl.ds(obase, RPW)])
        if with_cnt:
            pltpu.sync_copy(cnt_sh.at[pl.ds(row0, RPW)],
                            cnt_o.at[pl.ds(obase, RPW)])

    last = N - (NS - 1) * RPW

    @pl.when(s == NS - 1)
    def _():
        pltpu.sync_copy(acc.at[pl.ds(row0, last)], agg_o.at[pl.ds(obase, last)])
        if with_cnt:
            pltpu.sync_copy(cnt_sh.at[pl.ds(row0, last)],
                            cnt_o.at[pl.ds(obase, last)])


def _make_sc_agg(with_cnt, D):
    mesh = plsc.VectorSubcoreMesh(core_axis_name="c", subcore_axis_name="s",
                                  num_cores=NC, num_subcores=NS)
    out_type = [jax.ShapeDtypeStruct((NC * N, D), jnp.float32)]
    scratch = [
        pltpu.VMEM_SHARED((ACC_ROWS, D), jnp.float32),
    ]
    if with_cnt:
        out_type.append(jax.ShapeDtypeStruct((NC * N, CNTW), jnp.float32))
        scratch.append(pltpu.VMEM_SHARED((ACC_ROWS, CNTW), jnp.float32))
    scratch += [
        pltpu.VMEM((K,), jnp.int32),
        pltpu.VMEM((K,), jnp.int32),
        pltpu.VMEM((K,), jnp.int32),
        pltpu.VMEM((K,), jnp.int32),
        pltpu.VMEM((K, D // 2), jnp.int32),
        pltpu.VMEM((K, D // 2), jnp.int32),
        pltpu.VMEM((K, D), jnp.float32),
    ]
    if with_cnt:
        scratch.append(pltpu.VMEM((K, CNTW), jnp.float32))
    scratch += [pltpu.SemaphoreType.DMA] * 6

    return pl.kernel(
        functools.partial(_sc_agg_body, with_cnt, D),
        out_type=tuple(out_type),
        mesh=mesh,
        scratch_types=tuple(scratch),
        compiler_params=pltpu.CompilerParams(use_tc_tiling_on_sc=False),
        name=f"sc_agg_{D}_{int(with_cnt)}",
    )


BR = 2000
GRID = N // BR


def _pre_order(D):
    order = [0] * D
    for c in range(D // 32):
        for m in range(16):
            order[32 * c + 2 * m] = 32 * c + m
            order[32 * c + 2 * m + 1] = 32 * c + 16 + m
    return jnp.array(order, jnp.int32)


def _k1_body(xu, xi, wpu, bpu, wpi, bpi, wl0ui, wl0iu, wr0ui, wr0iu,
             z0u, z0i, r0u, r0i):
    hu = jax.nn.relu(jnp.dot(xu[...], wpu[...],
                             preferred_element_type=jnp.float32) + bpu[...])
    hi = jax.nn.relu(jnp.dot(xi[...], wpi[...],
                             preferred_element_type=jnp.float32) + bpi[...])
    z0u[...] = jnp.dot(hu, wl0ui[...],
                       preferred_element_type=jnp.float32).astype(jnp.bfloat16)
    z0i[...] = jnp.dot(hi, wl0iu[...],
                       preferred_element_type=jnp.float32).astype(jnp.bfloat16)
    r0u[...] = jnp.dot(hu, wr0iu[...], preferred_element_type=jnp.float32)
    r0i[...] = jnp.dot(hi, wr0ui[...], preferred_element_type=jnp.float32)


def _k2_body(agg_i, agg_u, cnt_i, cnt_u, r0u, r0i,
             bl0ui, bl0iu, g0u, be0u, g0i, be0i,
             wl1ui, wl1iu, wr1ui, wr1iu,
             z1u, z1i, r1u, r1i):
    mean_u = agg_u[...] / jnp.maximum(cnt_u[:, 0:1], 1.0)
    mean_i = agg_i[...] / jnp.maximum(cnt_i[:, 0:1], 1.0)
    tu = mean_u + bl0iu[...] + r0u[...]
    ti = mean_i + bl0ui[...] + r0i[...]
    hu = jax.nn.relu(tu * (g0u[...] * INV_STD) + be0u[...])
    hi = jax.nn.relu(ti * (g0i[...] * INV_STD) + be0i[...])
    z1u[...] = jnp.dot(hu, wl1ui[...],
                       preferred_element_type=jnp.float32).astype(jnp.bfloat16)
    z1i[...] = jnp.dot(hi, wl1iu[...],
                       preferred_element_type=jnp.float32).astype(jnp.bfloat16)
    r1u[...] = jnp.dot(hu, wr1iu[...], preferred_element_type=jnp.float32)
    r1i[...] = jnp.dot(hi, wr1ui[...], preferred_element_type=jnp.float32)


def _k3_body(agg_i, agg_u, cnt_i, cnt_u, r1u, r1i,
             bl1ui, bl1iu, g1u, be1u, g1i, be1i,
             hu_o, hi_o):
    mean_u = agg_u[...] / jnp.maximum(cnt_u[:, 0:1], 1.0)
    mean_i = agg_i[...] / jnp.maximum(cnt_i[:, 0:1], 1.0)
    tu = mean_u + bl1iu[...] + r1u[...]
    ti = mean_i + bl1ui[...] + r1i[...]
    hu_o[...] = jax.nn.relu(tu * (g1u[...] * INV_STD) + be1u[...])
    hi_o[...] = jax.nn.relu(ti * (g1i[...] * INV_STD) + be1i[...])


def _row_spec(w):
    return pl.BlockSpec((BR, w), lambda b: (b, 0))


def _row_spec_hi(w):
    return pl.BlockSpec((BR, w), lambda b: (b + GRID, 0))


def _full_spec(r, c):
    return pl.BlockSpec((r, c), lambda b: (0, 0))


def kernel(x_user, x_item, Wp_u, bp_u, Wp_i, bp_i,
           Wl0_ui, bl0_ui, Wr0_ui, Wl0_iu, bl0_iu, Wr0_iu,
           g0_u, be0_u, g0_i, be0_i,
           Wl1_ui, bl1_ui, Wr1_ui, Wl1_iu, bl1_iu, Wr1_iu,
           g1_u, be1_u, g1_i, be1_i,
           edge_index_ui, edge_index_iu):
    f32 = jnp.float32

    pad = EP - E
    src_ui = edge_index_ui[0].astype(jnp.int32)
    dst_ui = edge_index_ui[1].astype(jnp.int32)
    src_iu = edge_index_iu[0].astype(jnp.int32)
    dst_iu = edge_index_iu[1].astype(jnp.int32)
    zpad = jnp.zeros((pad,), jnp.int32)
    gpad = jnp.full((pad,), GARBAGE, jnp.int32)
    src_flat = jnp.concatenate([src_ui, zpad, src_iu + N, zpad + N])
    dst_flat = jnp.concatenate([dst_ui, gpad, dst_iu, gpad])

    row1 = lambda v: v.reshape(1, -1)

    k1 = pl.pallas_call(
        _k1_body,
        grid=(GRID,),
        in_specs=[
            _row_spec(D_IN), _row_spec(D_IN),
            _full_spec(D_IN, H), _full_spec(1, H),
            _full_spec(D_IN, H), _full_spec(1, H),
            _full_spec(H, H), _full_spec(H, H),
            _full_spec(H, H), _full_spec(H, H),
        ],
        out_specs=[_row_spec(H)] * 4,
        out_shape=[jax.ShapeDtypeStruct((N, H), jnp.bfloat16)] * 2
        + [jax.ShapeDtypeStruct((N, H), f32)] * 2,
    )
    p128 = _pre_order(H)
    p64 = _pre_order(OUT)
    z0u, z0i, r0u, r0i = k1(
        x_user, x_item, Wp_u.T, row1(bp_u), Wp_i.T, row1(bp_i),
        Wl0_ui.T[:, p128], Wl0_iu.T[:, p128], Wr0_ui.T, Wr0_iu.T)

    ztab0 = jax.lax.bitcast_convert_type(
        jnp.concatenate([z0u, z0i], axis=0).reshape(2 * N, H // 2, 2),
        jnp.int32)
    zacc_h = jnp.zeros((RPW, H), f32)
    zacc_o = jnp.zeros((RPW, OUT), f32)
    zcnt = jnp.zeros((RPW, CNTW), f32)
    ones_c = jnp.ones((K, CNTW), f32)
    agg0, cnt = _make_sc_agg(True, H)(ztab0, src_flat, dst_flat,
                                      zacc_h, zcnt, ones_c)

    k2 = pl.pallas_call(
        _k2_body,
        grid=(GRID,),
        in_specs=[
            _row_spec(H), _row_spec_hi(H),
            _row_spec(CNTW), _row_spec_hi(CNTW),
            _row_spec(H), _row_spec(H),
            _full_spec(1, H), _full_spec(1, H),
            _full_spec(1, H), _full_spec(1, H),
            _full_spec(1, H), _full_spec(1, H),
            _full_spec(H, OUT), _full_spec(H, OUT),
            _full_spec(H, OUT), _full_spec(H, OUT),
        ],
        out_specs=[_row_spec(OUT)] * 4,
        out_shape=[jax.ShapeDtypeStruct((N, OUT), jnp.bfloat16)] * 2
        + [jax.ShapeDtypeStruct((N, OUT), f32)] * 2,
    )
    z1u, z1i, r1u, r1i = k2(
        agg0, agg0, cnt, cnt, r0u, r0i,
        row1(bl0_ui), row1(bl0_iu), row1(g0_u), row1(be0_u),
        row1(g0_i), row1(be0_i),
        Wl1_ui.T[:, p64], Wl1_iu.T[:, p64], Wr1_ui.T, Wr1_iu.T)

    ztab1 = jax.lax.bitcast_convert_type(
        jnp.concatenate([z1u, z1i], axis=0).reshape(2 * N, OUT // 2, 2),
        jnp.int32)
    (agg1,) = _make_sc_agg(False, OUT)(ztab1, src_flat, dst_flat, zacc_o)

    k3 = pl.pallas_call(
        _k3_body,
        grid=(GRID,),
        in_specs=[
            _row_spec(OUT), _row_spec_hi(OUT),
            _row_spec(CNTW), _row_spec_hi(CNTW),
            _row_spec(OUT), _row_spec(OUT),
            _full_spec(1, OUT), _full_spec(1, OUT),
            _full_spec(1, OUT), _full_spec(1, OUT),
            _full_spec(1, OUT), _full_spec(1, OUT),
        ],
        out_specs=[_row_spec(OUT)] * 2,
        out_shape=[jax.ShapeDtypeStruct((N, OUT), f32)] * 2,
    )
    h_u, h_i = k3(
        agg1, agg1, cnt, cnt, r1u, r1i,
        row1(bl1_ui), row1(bl1_iu), row1(g1_u), row1(be1_u),
        row1(g1_i), row1(be1_i))
    return (h_u, h_i)


# speedup vs baseline: 1.3251x; 1.0901x over previous
import functools
import math

import jax
import jax.numpy as jnp
from jax import lax
from jax.experimental import pallas as pl
from jax.experimental.pallas import tpu as pltpu, tpu_sc as plsc

N = 10000
D_IN = 128
H = 128
OUT = 64
E = 320000
NC = 2
NS = 16
K = 128
CH = 160
EPW = CH * K
EP = NS * EPW
ACC_ROWS = 10240
RPW = ACC_ROWS // NS
GARBAGE = N
CNTW = 8
INV_STD = float(1.0 / math.sqrt(1.0 + 1e-5))


def _sc_agg_body(with_cnt, D, *refs):
    if with_cnt:
        (ztab, srcf, dstf, zacc_in, zcnt_in, ones_in, agg_o, cnt_o,
         acc, cnt_sh, sidx_a, sidx_b, didx_a, didx_b, rows_a, rows_b,
         ones, sem_a, sem_b, isem_a, isem_b, dsem_a, dsem_b) = refs
    else:
        (ztab, srcf, dstf, zacc_in, agg_o,
         acc, sidx_a, sidx_b, didx_a, didx_b, rows_a, rows_b,
         sem_a, sem_b, isem_a, isem_b, dsem_a, dsem_b) = refs
        cnt_o = cnt_sh = ones = None

    c = lax.axis_index("c")
    s = lax.axis_index("s")
    row0 = s * RPW
    base = c * EP + s * EPW

    pltpu.sync_copy(zacc_in, acc.at[pl.ds(row0, RPW)])
    if with_cnt:
        pltpu.sync_copy(zcnt_in, cnt_sh.at[pl.ds(row0, RPW)])
        pltpu.sync_copy(ones_in, ones)
    plsc.subcore_barrier()

    def sidx_start(j, sidx, isem):
        pltpu.async_copy(srcf.at[pl.ds(base + j * K, K)], sidx, isem)

    def sidx_wait(sidx, isem):
        pltpu.make_async_copy(srcf.at[pl.ds(0, K)], sidx, isem).wait()

    def didx_start(j, didx, dsem):
        pltpu.async_copy(dstf.at[pl.ds(base + j * K, K)], didx, dsem)

    def didx_wait(didx, dsem):
        pltpu.make_async_copy(dstf.at[pl.ds(0, K)], didx, dsem).wait()

    def gather_start(sidx, rows16, sem):
        pltpu.async_copy(ztab.at[sidx], rows16, sem)

    def gather_wait(sidx, rows16, sem):
        pltpu.make_async_copy(ztab.at[sidx], rows16, sem).wait()

    def scatter(didx, rows):
        pltpu.sync_copy(rows, acc.at[didx], add=True)
        if with_cnt:
            pltpu.sync_copy(ones, cnt_sh.at[didx], add=True)

    sidx_start(0, sidx_a, isem_a)
    didx_start(0, didx_a, dsem_a)
    sidx_start(1, sidx_b, isem_b)
    didx_start(1, didx_b, dsem_b)
    sidx_wait(sidx_a, isem_a)
    gather_start(sidx_a, rows_a, sem_a)

    def step(t, _):
        j0 = 2 * t
        sidx_wait(sidx_b, isem_b)
        gather_start(sidx_b, rows_b, sem_b)
        gather_wait(sidx_a, rows_a, sem_a)

        @pl.when(j0 + 2 < CH)
        def _():
            sidx_start(j0 + 2, sidx_a, isem_a)
        didx_wait(didx_a, dsem_a)
        scatter(didx_a, rows_a)

        @pl.when(j0 + 2 < CH)
        def _():
            didx_start(j0 + 2, didx_a, dsem_a)
            sidx_wait(sidx_a, isem_a)
            gather_start(sidx_a, rows_a, sem_a)
        gather_wait(sidx_b, rows_b, sem_b)

        @pl.when(j0 + 3 < CH)
        def _():
            sidx_start(j0 + 3, sidx_b, isem_b)
        didx_wait(didx_b, dsem_b)
        scatter(didx_b, rows_b)

        @pl.when(j0 + 3 < CH)
        def _():
            didx_start(j0 + 3, didx_b, dsem_b)
        return 0

    lax.fori_loop(0, CH // 2, step, 0)
    plsc.subcore_barrier()

    obase = c * N + row0

    @pl.when(s < NS - 1)
    def _():
        pltpu.sync_copy(acc.at[pl.ds(row0, RPW)], agg_o.at[pl.ds(obase, RPW)])
        if with_cnt:
            pltpu.sync_copy(cnt_sh.at[pl.ds(row0, RPW)],
                            cnt_o.at[pl.ds(obase, RPW)])

    last = N - (NS - 1) * RPW

    @pl.when(s == NS - 1)
    def _():
        pltpu.sync_copy(acc.at[pl.ds(row0, last)], agg_o.at[pl.ds(obase, last)])
        if with_cnt:
            pltpu.sync_copy(cnt_sh.at[pl.ds(row0, last)],
                            cnt_o.at[pl.ds(obase, last)])


def _make_sc_agg(with_cnt, D):
    mesh = plsc.VectorSubcoreMesh(core_axis_name="c", subcore_axis_name="s",
                                  num_cores=NC, num_subcores=NS)
    out_type = [jax.ShapeDtypeStruct((NC * N, D), jnp.float32)]
    scratch = [
        pltpu.VMEM_SHARED((ACC_ROWS, D), jnp.float32),
    ]
    if with_cnt:
        out_type.append(jax.ShapeDtypeStruct((NC * N, CNTW), jnp.float32))
        scratch.append(pltpu.VMEM_SHARED((ACC_ROWS, CNTW), jnp.float32))
    scratch += [
        pltpu.VMEM((K,), jnp.int32),
        pltpu.VMEM((K,), jnp.int32),
        pltpu.VMEM((K,), jnp.int32),
        pltpu.VMEM((K,), jnp.int32),
        pltpu.VMEM((K, D), jnp.float32),
        pltpu.VMEM((K, D), jnp.float32),
    ]
    if with_cnt:
        scratch.append(pltpu.VMEM((K, CNTW), jnp.float32))
    scratch += [pltpu.SemaphoreType.DMA] * 6

    return pl.kernel(
        functools.partial(_sc_agg_body, with_cnt, D),
        out_type=tuple(out_type),
        mesh=mesh,
        scratch_types=tuple(scratch),
        compiler_params=pltpu.CompilerParams(use_tc_tiling_on_sc=False),
        name=f"sc_agg_{D}_{int(with_cnt)}",
    )


BR = 2000
GRID = N // BR


def _k1_body(xu, xi, wpu, bpu, wpi, bpi, wl0ui, wl0iu, wr0ui, wr0iu,
             z0u, z0i, r0u, r0i):
    hu = jax.nn.relu(jnp.dot(xu[...], wpu[...],
                             preferred_element_type=jnp.float32) + bpu[...])
    hi = jax.nn.relu(jnp.dot(xi[...], wpi[...],
                             preferred_element_type=jnp.float32) + bpi[...])
    z0u[...] = jnp.dot(hu, wl0ui[...], preferred_element_type=jnp.float32)
    z0i[...] = jnp.dot(hi, wl0iu[...], preferred_element_type=jnp.float32)
    r0u[...] = jnp.dot(hu, wr0iu[...], preferred_element_type=jnp.float32)
    r0i[...] = jnp.dot(hi, wr0ui[...], preferred_element_type=jnp.float32)


def _k2_body(agg_i, agg_u, cnt_i, cnt_u, r0u, r0i,
             bl0ui, bl0iu, g0u, be0u, g0i, be0i,
             wl1ui, wl1iu, wr1ui, wr1iu,
             z1u, z1i, r1u, r1i):
    mean_u = agg_u[...] / jnp.maximum(cnt_u[:, 0:1], 1.0)
    mean_i = agg_i[...] / jnp.maximum(cnt_i[:, 0:1], 1.0)
    tu = mean_u + bl0iu[...] + r0u[...]
    ti = mean_i + bl0ui[...] + r0i[...]
    hu = jax.nn.relu(tu * (g0u[...] * INV_STD) + be0u[...])
    hi = jax.nn.relu(ti * (g0i[...] * INV_STD) + be0i[...])
    z1u[...] = jnp.dot(hu, wl1ui[...], preferred_element_type=jnp.float32)
    z1i[...] = jnp.dot(hi, wl1iu[...], preferred_element_type=jnp.float32)
    r1u[...] = jnp.dot(hu, wr1iu[...], preferred_element_type=jnp.float32)
    r1i[...] = jnp.dot(hi, wr1ui[...], preferred_element_type=jnp.float32)


def _k3_body(agg_i, agg_u, cnt_i, cnt_u, r1u, r1i,
             bl1ui, bl1iu, g1u, be1u, g1i, be1i,
             hu_o, hi_o):
    mean_u = agg_u[...] / jnp.maximum(cnt_u[:, 0:1], 1.0)
    mean_i = agg_i[...] / jnp.maximum(cnt_i[:, 0:1], 1.0)
    tu = mean_u + bl1iu[...] + r1u[...]
    ti = mean_i + bl1ui[...] + r1i[...]
    hu_o[...] = jax.nn.relu(tu * (g1u[...] * INV_STD) + be1u[...])
    hi_o[...] = jax.nn.relu(ti * (g1i[...] * INV_STD) + be1i[...])


def _row_spec(w):
    return pl.BlockSpec((BR, w), lambda b: (b, 0))


def _row_spec_hi(w):
    return pl.BlockSpec((BR, w), lambda b: (b + GRID, 0))


def _full_spec(r, c):
    return pl.BlockSpec((r, c), lambda b: (0, 0))


def kernel(x_user, x_item, Wp_u, bp_u, Wp_i, bp_i,
           Wl0_ui, bl0_ui, Wr0_ui, Wl0_iu, bl0_iu, Wr0_iu,
           g0_u, be0_u, g0_i, be0_i,
           Wl1_ui, bl1_ui, Wr1_ui, Wl1_iu, bl1_iu, Wr1_iu,
           g1_u, be1_u, g1_i, be1_i,
           edge_index_ui, edge_index_iu):
    f32 = jnp.float32

    pad = EP - E
    src_ui = edge_index_ui[0].astype(jnp.int32)
    dst_ui = edge_index_ui[1].astype(jnp.int32)
    src_iu = edge_index_iu[0].astype(jnp.int32)
    dst_iu = edge_index_iu[1].astype(jnp.int32)
    zpad = jnp.zeros((pad,), jnp.int32)
    gpad = jnp.full((pad,), GARBAGE, jnp.int32)
    src_flat = jnp.concatenate([src_ui, zpad, src_iu + N, zpad + N])
    dst_flat = jnp.concatenate([dst_ui, gpad, dst_iu, gpad])

    row1 = lambda v: v.reshape(1, -1)

    k1 = pl.pallas_call(
        _k1_body,
        grid=(GRID,),
        in_specs=[
            _row_spec(D_IN), _row_spec(D_IN),
            _full_spec(D_IN, H), _full_spec(1, H),
            _full_spec(D_IN, H), _full_spec(1, H),
            _full_spec(H, H), _full_spec(H, H),
            _full_spec(H, H), _full_spec(H, H),
        ],
        out_specs=[_row_spec(H)] * 4,
        out_shape=[jax.ShapeDtypeStruct((N, H), f32)] * 4,
    )
    z0u, z0i, r0u, r0i = k1(
        x_user, x_item, Wp_u.T, row1(bp_u), Wp_i.T, row1(bp_i),
        Wl0_ui.T, Wl0_iu.T, Wr0_ui.T, Wr0_iu.T)

    ztab0 = jnp.concatenate([z0u, z0i], axis=0)
    zacc_h = jnp.zeros((RPW, H), f32)
    zacc_o = jnp.zeros((RPW, OUT), f32)
    zcnt = jnp.zeros((RPW, CNTW), f32)
    ones_c = jnp.ones((K, CNTW), f32)
    agg0, cnt = _make_sc_agg(True, H)(ztab0, src_flat, dst_flat,
                                      zacc_h, zcnt, ones_c)

    k2 = pl.pallas_call(
        _k2_body,
        grid=(GRID,),
        in_specs=[
            _row_spec(H), _row_spec_hi(H),
            _row_spec(CNTW), _row_spec_hi(CNTW),
            _row_spec(H), _row_spec(H),
            _full_spec(1, H), _full_spec(1, H),
            _full_spec(1, H), _full_spec(1, H),
            _full_spec(1, H), _full_spec(1, H),
            _full_spec(H, OUT), _full_spec(H, OUT),
            _full_spec(H, OUT), _full_spec(H, OUT),
        ],
        out_specs=[_row_spec(OUT)] * 4,
        out_shape=[jax.ShapeDtypeStruct((N, OUT), f32)] * 4,
    )
    z1u, z1i, r1u, r1i = k2(
        agg0, agg0, cnt, cnt, r0u, r0i,
        row1(bl0_ui), row1(bl0_iu), row1(g0_u), row1(be0_u),
        row1(g0_i), row1(be0_i),
        Wl1_ui.T, Wl1_iu.T, Wr1_ui.T, Wr1_iu.T)

    ztab1 = jnp.concatenate([z1u, z1i], axis=0)
    (agg1,) = _make_sc_agg(False, OUT)(ztab1, src_flat, dst_flat, zacc_o)

    k3 = pl.pallas_call(
        _k3_body,
        grid=(GRID,),
        in_specs=[
            _row_spec(OUT), _row_spec_hi(OUT),
            _row_spec(CNTW), _row_spec_hi(CNTW),
            _row_spec(OUT), _row_spec(OUT),
            _full_spec(1, OUT), _full_spec(1, OUT),
            _full_spec(1, OUT), _full_spec(1, OUT),
            _full_spec(1, OUT), _full_spec(1, OUT),
        ],
        out_specs=[_row_spec(OUT)] * 2,
        out_shape=[jax.ShapeDtypeStruct((N, OUT), f32)] * 2,
    )
    h_u, h_i = k3(
        agg1, agg1, cnt, cnt, r1u, r1i,
        row1(bl1_ui), row1(bl1_iu), row1(g1_u), row1(be1_u),
        row1(g1_i), row1(be1_i))
    return (h_u, h_i)
